# Initial kernel scaffold; baseline (speedup 1.0000x reference)
import jax
import jax.numpy as jnp
from jax.experimental import pallas as pl


def kernel(pos, v, z, Wr, Wmix, w_embed, w_vinit, w_out):
    raise NotImplementedError("write your pallas kernel here")


# speedup vs baseline: 24.8675x; 24.8675x over previous
import jax
import jax.numpy as jnp
import numpy as np
from jax.experimental import pallas as pl

B = 1024
N = 20
D = 3
NF = 16
N_RBF = 16
N_LAYERS = 3
G = 8


def _tfn_block(pos_ref, v_ref, zf_ref, wrcat_ref, wmix_ref, wemb_ref,
               wvin_ref, wout_ref, cen_ref, mask_ref, out_ref):
    centers = cen_ref[0]
    p = pos_ref[...]
    vv = v_ref[...]
    zf = zf_ref[...]
    d = p[:, None, :, :] - p[:, :, None, :]
    r2 = jnp.sum(d * d, axis=-1) + 1e-8
    r = jnp.sqrt(r2)
    inv_r = 1.0 / r
    dhat = [d[..., k] * inv_r for k in range(D)]
    mask = mask_ref[...][None, :, :, None]
    rbf = jnp.exp(-2.0 * (r[..., None] - centers) ** 2) * mask
    wall = rbf.reshape(G * N * N, N_RBF) @ wrcat_ref[...]

    f0 = zf[..., None] * wemb_ref[0]
    f1 = [vv[..., k][..., None] * wvin_ref[0] for k in range(D)]

    for l in range(N_LAYERS):
        def w_(kind):
            s = (3 * l + kind) * NF
            return wall[:, s:s + NF].reshape(G, N, N, NF)
        w0, w1, w2 = w_(0), w_(1), w_(2)
        a0 = jnp.sum(w0 * f0[:, :, None, :], axis=1)
        a1 = [jnp.sum(w1 * f1[k][:, :, None, :] + w2 * dhat[k][..., None], axis=1)
              for k in range(D)]
        f0 = jax.nn.relu(
            (a0.reshape(G * N, NF) @ wmix_ref[l, 0]
             + f0.reshape(G * N, NF) @ wmix_ref[l, 1]).reshape(G, N, NF))
        f1 = [(a1[k].reshape(G * N, NF) @ wmix_ref[l, 2]
               + f1[k].reshape(G * N, NF) @ wmix_ref[l, 3]).reshape(G, N, NF)
              for k in range(D)]

    out1 = jnp.stack([jnp.sum(f1[k] * wout_ref[0], axis=-1) for k in range(D)],
                     axis=-1)
    out_ref[...] = out1 + p


@jax.jit
def kernel(pos, v, z, Wr, Wmix, w_embed, w_vinit, w_out):
    posr = pos.reshape(B, N, D)
    vr = v.reshape(B, N, D)
    zf = z.astype(jnp.float32).reshape(B, N)
    wrcat = Wr.transpose(2, 0, 1, 3).reshape(N_RBF, N_LAYERS * 3 * NF)
    grid = (B // G,)
    full = lambda *s: pl.BlockSpec(s, lambda i: (0,) * len(s))
    out = pl.pallas_call(
        _tfn_block,
        grid=grid,
        in_specs=[
            pl.BlockSpec((G, N, D), lambda i: (i, 0, 0)),
            pl.BlockSpec((G, N, D), lambda i: (i, 0, 0)),
            pl.BlockSpec((G, N), lambda i: (i, 0)),
            full(N_RBF, N_LAYERS * 3 * NF),
            full(N_LAYERS, 4, NF, NF),
            full(1, NF),
            full(1, NF),
            full(1, NF),
            full(1, N_RBF),
            full(N, N),
        ],
        out_specs=pl.BlockSpec((G, N, D), lambda i: (i, 0, 0)),
        out_shape=jax.ShapeDtypeStruct((B, N, D), jnp.float32),
    )(posr, vr, zf, wrcat, Wmix, w_embed.reshape(1, NF),
      w_vinit.reshape(1, NF), w_out.reshape(1, NF),
      jnp.asarray(np.linspace(0.0, 4.0, N_RBF, dtype=np.float32)[None, :]),
      jnp.asarray(1.0 - np.eye(N, dtype=np.float32)))
    return out.reshape(B * N, D)
